# baseline (speedup 1.0000x reference)
import functools

import jax
import jax.numpy as jnp
from jax import lax
from jax.experimental import pallas as pl
from jax.experimental.pallas import tpu as pltpu
from jax.experimental.pallas import tpu_sc as plsc

NC = 2
NS = 16
NW = NC * NS

L = 16
D = 10
NB = 16384
NL = 200
BSTRIP = NB // NW
N_LT = NL // 8
CVECS = BSTRIP // L


def _sc_embed(x_t, table):
    mesh = plsc.VectorSubcoreMesh(
        core_axis_name="c", subcore_axis_name="s", num_cores=NC, num_subcores=NS
    )

    @functools.partial(
        pl.kernel,
        mesh=mesh,
        out_type=jax.ShapeDtypeStruct((D, NL, NB), jnp.float32),
        scratch_types=[
            pltpu.VMEM((D * D, L), jnp.float32),
            pltpu.VMEM((2, 8, BSTRIP), jnp.int32),
            pltpu.VMEM((2, D, 8, BSTRIP), jnp.float32),
            pltpu.SemaphoreType.DMA,
            pltpu.SemaphoreType.DMA,
            pltpu.SemaphoreType.DMA,
            pltpu.SemaphoreType.DMA,
        ],
        compiler_params=pltpu.CompilerParams(needs_layout_passes=False),
    )
    def k(x_hbm, table_hbm, out_hbm, table_v, idx_v, out_v,
          sem_in0, sem_in1, sem_out0, sem_out1):
        wid = lax.axis_index("s") * NC + lax.axis_index("c")
        b0 = wid * BSTRIP
        sems_in = (sem_in0, sem_in1)
        sems_out = (sem_out0, sem_out1)
        pltpu.sync_copy(table_hbm, table_v)
        lane = lax.iota(jnp.int32, L)

        def in_desc(lt, b):
            return pltpu.make_async_copy(
                x_hbm.at[pl.ds(lt * 8, 8), pl.ds(b0, BSTRIP)],
                idx_v.at[b], sems_in[b],
            )

        def out_desc(lt, b):
            return pltpu.make_async_copy(
                out_v.at[b],
                out_hbm.at[:, pl.ds(lt * 8, 8), pl.ds(b0, BSTRIP)],
                sems_out[b],
            )

        def compute(b):
            @plsc.parallel_loop(0, CVECS, unroll=4)
            def c_body(c):
                for s in range(8):
                    idx_vec = idx_v[b, s, pl.ds(c * L, L)]
                    e_base = idx_vec * D
                    for j in range(D):
                        vals = plsc.load_gather(table_v, [e_base + j, lane])
                        out_v[b, j, s, pl.ds(c * L, L)] = vals

        in_desc(0, 0).start()
        in_desc(1, 1).start()

        def body(kk, carry):
            for b in range(2):
                lt = 2 * kk + b
                in_desc(0, b).wait()

                @pl.when(kk > 0)
                def _():
                    out_desc(0, b).wait()

                compute(b)
                out_desc(lt, b).start()

                nxt = lt + 2
                if b == 0:
                    in_desc(nxt, b).start()
                else:
                    @pl.when(kk < 11)
                    def _():
                        in_desc(nxt, b).start()
            return carry

        lax.fori_loop(0, 12, body, 0)

        in_desc(0, 0).wait()
        out_desc(0, 0).wait()
        compute(0)
        out_desc(24, 0).start()
        out_desc(0, 0).wait()
        out_desc(0, 1).wait()

    return k(x_t, table)


def kernel(x, embedding_weight):
    x_t = jnp.swapaxes(x, 0, 1).astype(jnp.int32)
    w_rep = jnp.broadcast_to(
        embedding_weight.astype(jnp.float32).reshape(D * D, 1), (D * D, L)
    )
    out_t = _sc_embed(x_t, w_rep)
    return jnp.transpose(out_t, (2, 1, 0))


# speedup vs baseline: 1.7172x; 1.7172x over previous
import functools

import jax
import jax.numpy as jnp
from jax import lax
from jax.experimental import pallas as pl
from jax.experimental.pallas import tpu as pltpu
from jax.experimental.pallas import tpu_sc as plsc

NC = 2
NS = 16
NW = NC * NS

L = 16
D = 10
NB = 16384
NL = 200
BSTRIP = NB // NW
N_LT = NL // 8
CVECS = BSTRIP // L


def _sc_embed(x_t, table):
    mesh = plsc.VectorSubcoreMesh(
        core_axis_name="c", subcore_axis_name="s", num_cores=NC, num_subcores=NS
    )

    @functools.partial(
        pl.kernel,
        mesh=mesh,
        out_type=jax.ShapeDtypeStruct((D, NL, NB), jnp.float32),
        scratch_types=[
            pltpu.VMEM((D * D, L), jnp.float32),
            pltpu.VMEM((2, 8, BSTRIP), jnp.int32),
            pltpu.VMEM((2, D, 8, BSTRIP), jnp.float32),
            pltpu.SemaphoreType.DMA,
            pltpu.SemaphoreType.DMA,
            pltpu.SemaphoreType.DMA,
            pltpu.SemaphoreType.DMA,
        ],
        compiler_params=pltpu.CompilerParams(needs_layout_passes=False),
    )
    def k(x_hbm, table_hbm, out_hbm, table_v, idx_v, out_v,
          sem_in0, sem_in1, sem_out0, sem_out1):
        wid = lax.axis_index("s") * NC + lax.axis_index("c")
        b0 = wid * BSTRIP
        sems_in = (sem_in0, sem_in1)
        sems_out = (sem_out0, sem_out1)
        pltpu.sync_copy(table_hbm, table_v)
        lane = lax.iota(jnp.int32, L)

        def in_desc(lt, b):
            return pltpu.make_async_copy(
                x_hbm.at[pl.ds(lt * 8, 8), pl.ds(b0, BSTRIP)],
                idx_v.at[b], sems_in[b],
            )

        def out_desc(lt, b):
            return pltpu.make_async_copy(
                out_v.at[b],
                out_hbm.at[:, pl.ds(lt * 8, 8), pl.ds(b0, BSTRIP)],
                sems_out[b],
            )

        def compute(b):
            @plsc.parallel_loop(0, CVECS, unroll=1)
            def c_body(c):
                for s in range(8):
                    idx_vec = idx_v[b, s, pl.ds(c * L, L)]
                    e_base = idx_vec * D
                    for j in range(D):
                        vals = plsc.load_gather(table_v, [e_base + j, lane])
                        out_v[b, j, s, pl.ds(c * L, L)] = vals

        in_desc(0, 0).start()
        in_desc(1, 1).start()

        def body(kk, carry):
            for b in range(2):
                lt = 2 * kk + b
                in_desc(0, b).wait()

                @pl.when(kk > 0)
                def _():
                    out_desc(0, b).wait()

                compute(b)
                out_desc(lt, b).start()

                nxt = lt + 2
                if b == 0:
                    in_desc(nxt, b).start()
                else:
                    @pl.when(kk < 11)
                    def _():
                        in_desc(nxt, b).start()
            return carry

        lax.fori_loop(0, 12, body, 0)

        in_desc(0, 0).wait()
        out_desc(0, 0).wait()
        compute(0)
        out_desc(24, 0).start()
        out_desc(0, 0).wait()
        out_desc(0, 1).wait()

    return k(x_t, table)


def kernel(x, embedding_weight):
    x_t = jnp.swapaxes(x, 0, 1).astype(jnp.int32)
    w_rep = jnp.broadcast_to(
        embedding_weight.astype(jnp.float32).reshape(D * D, 1), (D * D, L)
    )
    out_t = _sc_embed(x_t, w_rep)
    return jnp.transpose(out_t, (2, 1, 0))
